# baseline (speedup 1.0000x reference)
import jax
import jax.numpy as jnp
from jax import lax
from jax.experimental import pallas as pl
from jax.experimental.pallas import tpu as pltpu
from jax.experimental.pallas import tpu_sc as plsc

_NC = 8192
_NV = 6000
_NF = 8192
_RAD = 0.003
_GRAV = 9.81
_EPS = 1e-4

_NCORES = 2
_NSUB = 16
_NW = _NCORES * _NSUB
_FPT = _NF // _NW
_PPT = _NC // _NW

_MESH = dict(core_axis_name="c", subcore_axis_name="s",
             num_cores=_NCORES, num_subcores=_NSUB)
_SC_PARAMS = pltpu.CompilerParams(use_tc_tiling_on_sc=False,
                                  needs_layout_passes=False)


def _sqrt16(x):
    i = lax.bitcast_convert_type(x, jnp.int32)
    y = lax.bitcast_convert_type(jnp.int32(0x5F3759DF) - (i >> 1), jnp.float32)
    for _ in range(3):
        y = y * (1.5 - 0.5 * x * y * y)
    return x * y


def _face_body(vt_hbm, f0_hbm, f1_hbm, f2_hbm, tbl_hbm,
               vt_v, f0_v, f1_v, f2_v, o_v):
    c = lax.axis_index("c")
    s = lax.axis_index("s")
    wid = s * _NCORES + c
    base = wid * _FPT
    pltpu.sync_copy(vt_hbm, vt_v)
    pltpu.sync_copy(f0_hbm.at[pl.ds(base, _FPT)], f0_v)
    pltpu.sync_copy(f1_hbm.at[pl.ds(base, _FPT)], f1_v)
    pltpu.sync_copy(f2_hbm.at[pl.ds(base, _FPT)], f2_v)

    def g(row, idx):
        return plsc.load_gather(vt_v, [jnp.full((16,), row, jnp.int32), idx])

    for k in range(_FPT // 16):
        o = k * 16
        i0 = f0_v[pl.ds(o, 16)]
        i1 = f1_v[pl.ds(o, 16)]
        i2 = f2_v[pl.ds(o, 16)]
        for t in range(2):
            r0 = 3 * t
            v0 = [g(r0 + d, i0) for d in range(3)]
            v1 = [g(r0 + d, i1) for d in range(3)]
            v2 = [g(r0 + d, i2) for d in range(3)]
            for d in range(3):
                o_v[3 * t + d, pl.ds(o, 16)] = (v0[d] + v1[d] + v2[d]) * (1.0 / 3.0)
            e1 = [v1[d] - v0[d] for d in range(3)]
            e2 = [v2[d] - v0[d] for d in range(3)]
            o_v[6 + r0 + 0, pl.ds(o, 16)] = e1[1] * e2[2] - e1[2] * e2[1]
            o_v[6 + r0 + 1, pl.ds(o, 16)] = e1[2] * e2[0] - e1[0] * e2[2]
            o_v[6 + r0 + 2, pl.ds(o, 16)] = e1[0] * e2[1] - e1[1] * e2[0]
    pltpu.sync_copy(o_v, tbl_hbm.at[:, pl.ds(base, _FPT)])


_face_kernel = pl.kernel(
    _face_body,
    out_type=jax.ShapeDtypeStruct((12, _NF), jnp.float32),
    mesh=plsc.VectorSubcoreMesh(**_MESH),
    scratch_types=[
        pltpu.VMEM((6, _NV), jnp.float32),
        pltpu.VMEM((_FPT,), jnp.int32),
        pltpu.VMEM((_FPT,), jnp.int32),
        pltpu.VMEM((_FPT,), jnp.int32),
        pltpu.VMEM((12, _FPT), jnp.float32),
    ],
    compiler_params=_SC_PARAMS,
)


_BQ = 512
_CW = 512


_NBLK = _NC // _BQ


def _knn_body(q_ref, tbl_ref, d_ref, i_ref):
    q = q_ref[...]
    qx, qy, qz = q[:, 0:1], q[:, 1:2], q[:, 2:3]
    kt = tbl_ref[0]
    kx, ky, kz = kt[0:1, :], kt[1:2, :], kt[2:3, :]
    psq = qx * qx + qy * qy + qz * qz
    ksq = kx * kx + ky * ky + kz * kz
    g = jnp.dot(q.astype(jnp.bfloat16), kt.astype(jnp.bfloat16),
                preferred_element_type=jnp.float32)
    runmin = jnp.full((_BQ, _CW), jnp.inf, jnp.float32)
    runchk = jnp.zeros((_BQ, _CW), jnp.float32)
    for t in range(_NF // _CW):
        sl = slice(t * _CW, (t + 1) * _CW)
        dt = psq + ksq[:, sl] - 2.0 * g[:, sl]
        c = dt < runmin
        runmin = jnp.where(c, dt, runmin)
        runchk = jnp.where(c, jnp.float32(t), runchk)
    gmin = jnp.min(runmin, axis=1, keepdims=True)
    lane = lax.broadcasted_iota(jnp.int32, (_BQ, _CW), 1).astype(jnp.float32)
    cand = jnp.where(runmin <= gmin, runchk * jnp.float32(_CW) + lane,
                     jnp.float32(_NF))
    idxf = jnp.min(cand, axis=1)
    d_ref[...] = gmin[:, 0]
    i_ref[...] = idxf.astype(jnp.int32)


_knn_kernel = pl.pallas_call(
    _knn_body,
    grid=(2 * _NBLK,),
    in_specs=[
        pl.BlockSpec((_BQ, 3), lambda i: (i, 0)),
        pl.BlockSpec((1, 3, _NF), lambda i: (i // _NBLK, 0, 0)),
    ],
    out_specs=[
        pl.BlockSpec((_BQ,), lambda i: (i,)),
        pl.BlockSpec((_BQ,), lambda i: (i,)),
    ],
    out_shape=[
        jax.ShapeDtypeStruct((2 * _NC,), jnp.float32),
        jax.ShapeDtypeStruct((2 * _NC,), jnp.int32),
    ],
    compiler_params=pltpu.CompilerParams(vmem_limit_bytes=128 * 1024 * 1024),
)


def _friction_body(tbl_hbm, ip_hbm, ic_hbm, dp_hbm, dc_hbm,
                   cpos_hbm, cpred_hbm, m_hbm, out_hbm,
                   tbl_v, ip_v, ic_v, dp_v, dc_v, cpos_v, cpred_v, m_v,
                   acc_v, shared_v, red_v, out_v):
    c = lax.axis_index("c")
    s = lax.axis_index("s")
    wid = s * _NCORES + c
    base = wid * _PPT
    pltpu.sync_copy(tbl_hbm, tbl_v)
    pltpu.sync_copy(ip_hbm.at[pl.ds(base, _PPT)], ip_v)
    pltpu.sync_copy(ic_hbm.at[pl.ds(base, _PPT)], ic_v)
    pltpu.sync_copy(dp_hbm.at[pl.ds(base, _PPT)], dp_v)
    pltpu.sync_copy(dc_hbm.at[pl.ds(base, _PPT)], dc_v)
    pltpu.sync_copy(cpos_hbm.at[:, pl.ds(base, _PPT)], cpos_v)
    pltpu.sync_copy(cpred_hbm.at[:, pl.ds(base, _PPT)], cpred_v)
    pltpu.sync_copy(m_hbm.at[pl.ds(base, _PPT)], m_v)

    def g(row, idx):
        return plsc.load_gather(tbl_v, [jnp.full((16,), row, jnp.int32), idx])

    def unit(n):
        s2 = n[0] * n[0] + n[1] * n[1] + n[2] * n[2]
        inv = 1.0 / jnp.maximum(_sqrt16(s2), 1e-12)
        return [n[0] * inv, n[1] * inv, n[2] * inv]

    acc = jnp.zeros((16,), jnp.float32)
    for k in range(_PPT // 16):
        o = k * 16
        jp = ip_v[pl.ds(o, 16)]
        jc = ic_v[pl.ds(o, 16)]
        pp = [g(r, jp) for r in (0, 1, 2)]
        pc = [g(r, jp) for r in (3, 4, 5)]
        nmu = unit([g(r, jp) for r in (6, 7, 8)])
        ncu = unit([g(r, jc) for r in (9, 10, 11)])
        nm = [(nmu[d] + ncu[d]) * 0.5 for d in range(3)]
        dl = [cpred_v[d, pl.ds(o, 16)] - (cpos_v[d, pl.ds(o, 16)] + (pc[d] - pp[d]))
              for d in range(3)]
        t = dl[0] * nm[0] + dl[1] * nm[1] + dl[2] * nm[2]
        dproj = [dl[d] - t * nm[d] for d in range(3)]
        d2 = dproj[0] * dproj[0] + dproj[1] * dproj[1] + dproj[2] * dproj[2]
        dn = _sqrt16(d2)
        cosn = jnp.abs(dproj[2]) / jnp.maximum(dn, 1e-8)
        cosp = _sqrt16(jnp.abs(1.0 - cosn * cosn) + _EPS)
        mask = jnp.logical_and(dp_v[pl.ds(o, 16)] < _RAD,
                               dc_v[pl.ds(o, 16)] < _RAD)
        fr = m_v[pl.ds(o, 16)] * cosp * _GRAV * dn * jnp.where(mask, 1.0, 0.0)
        acc = acc + fr
    acc_v[...] = acc
    pltpu.sync_copy(acc_v, shared_v.at[s])
    plsc.subcore_barrier()

    @pl.when(s == 0)
    def _():
        pltpu.sync_copy(shared_v, red_v)
        tot = jnp.zeros((16,), jnp.float32)
        for r in range(_NSUB):
            tot = tot + red_v[r]
        out_v[...] = jnp.full((16,), jnp.sum(tot), jnp.float32)
        pltpu.sync_copy(out_v, out_hbm.at[c])


_friction_kernel = pl.kernel(
    _friction_body,
    out_type=jax.ShapeDtypeStruct((_NCORES, 16), jnp.float32),
    mesh=plsc.VectorSubcoreMesh(**_MESH),
    scratch_types=[
        pltpu.VMEM((12, _NF), jnp.float32),
        pltpu.VMEM((_PPT,), jnp.int32),
        pltpu.VMEM((_PPT,), jnp.int32),
        pltpu.VMEM((_PPT,), jnp.float32),
        pltpu.VMEM((_PPT,), jnp.float32),
        pltpu.VMEM((3, _PPT), jnp.float32),
        pltpu.VMEM((3, _PPT), jnp.float32),
        pltpu.VMEM((_PPT,), jnp.float32),
        pltpu.VMEM((16,), jnp.float32),
        pltpu.VMEM_SHARED((_NSUB, 16), jnp.float32),
        pltpu.VMEM((_NSUB, 16), jnp.float32),
        pltpu.VMEM((16,), jnp.float32),
    ],
    compiler_params=_SC_PARAMS,
)


def kernel(cloth_pos, cloth_pred_pos, cloth_v_mass,
           obstacle_prev_pos, obstacle_pos, obstacle_faces):
    vt = jnp.concatenate([obstacle_prev_pos.T, obstacle_pos.T], axis=0)
    tbl = _face_kernel(vt, obstacle_faces[0], obstacle_faces[1],
                       obstacle_faces[2])
    q2 = jnp.concatenate([cloth_pos, cloth_pred_pos], axis=0)
    dist2, idx2 = _knn_kernel(q2, tbl[:6].reshape(2, 3, _NF))
    dp, dc = dist2[:_NC], dist2[_NC:]
    ip, ic = idx2[:_NC], idx2[_NC:]
    part = _friction_kernel(tbl, ip, ic, dp, dc,
                            cloth_pos.T, cloth_pred_pos.T,
                            cloth_v_mass[:, 0])
    return part[0, 0] + part[1, 0]


# speedup vs baseline: 1.1509x; 1.1509x over previous
import jax
import jax.numpy as jnp
from jax import lax
from jax.experimental import pallas as pl
from jax.experimental.pallas import tpu as pltpu
from jax.experimental.pallas import tpu_sc as plsc

_NC = 8192
_NV = 6000
_NF = 8192
_RAD = 0.003
_GRAV = 9.81
_EPS = 1e-4

_NCORES = 2
_NSUB = 16
_NW = _NCORES * _NSUB
_FPT = _NF // _NW
_PPT = _NC // _NW

_MESH = dict(core_axis_name="c", subcore_axis_name="s",
             num_cores=_NCORES, num_subcores=_NSUB)
_SC_PARAMS = pltpu.CompilerParams(use_tc_tiling_on_sc=False,
                                  needs_layout_passes=False)


def _sqrt16(x):
    i = lax.bitcast_convert_type(x, jnp.int32)
    y = lax.bitcast_convert_type(jnp.int32(0x5F3759DF) - (i >> 1), jnp.float32)
    for _ in range(3):
        y = y * (1.5 - 0.5 * x * y * y)
    return x * y


def _face_body(vt_hbm, f0_hbm, f1_hbm, f2_hbm, tbl_hbm,
               vt_v, f0_v, f1_v, f2_v, o_v):
    c = lax.axis_index("c")
    s = lax.axis_index("s")
    wid = s * _NCORES + c
    base = wid * _FPT
    pltpu.sync_copy(vt_hbm, vt_v)
    pltpu.sync_copy(f0_hbm.at[pl.ds(base, _FPT)], f0_v)
    pltpu.sync_copy(f1_hbm.at[pl.ds(base, _FPT)], f1_v)
    pltpu.sync_copy(f2_hbm.at[pl.ds(base, _FPT)], f2_v)

    def g(row, idx):
        return plsc.load_gather(vt_v, [jnp.full((16,), row, jnp.int32), idx])

    for k in range(_FPT // 16):
        o = k * 16
        i0 = f0_v[pl.ds(o, 16)]
        i1 = f1_v[pl.ds(o, 16)]
        i2 = f2_v[pl.ds(o, 16)]
        for t in range(2):
            r0 = 3 * t
            v0 = [g(r0 + d, i0) for d in range(3)]
            v1 = [g(r0 + d, i1) for d in range(3)]
            v2 = [g(r0 + d, i2) for d in range(3)]
            for d in range(3):
                o_v[3 * t + d, pl.ds(o, 16)] = (v0[d] + v1[d] + v2[d]) * (1.0 / 3.0)
            e1 = [v1[d] - v0[d] for d in range(3)]
            e2 = [v2[d] - v0[d] for d in range(3)]
            o_v[6 + r0 + 0, pl.ds(o, 16)] = e1[1] * e2[2] - e1[2] * e2[1]
            o_v[6 + r0 + 1, pl.ds(o, 16)] = e1[2] * e2[0] - e1[0] * e2[2]
            o_v[6 + r0 + 2, pl.ds(o, 16)] = e1[0] * e2[1] - e1[1] * e2[0]
    pltpu.sync_copy(o_v, tbl_hbm.at[:, pl.ds(base, _FPT)])


_face_kernel = pl.kernel(
    _face_body,
    out_type=jax.ShapeDtypeStruct((12, _NF), jnp.float32),
    mesh=plsc.VectorSubcoreMesh(**_MESH),
    scratch_types=[
        pltpu.VMEM((6, _NV), jnp.float32),
        pltpu.VMEM((_FPT,), jnp.int32),
        pltpu.VMEM((_FPT,), jnp.int32),
        pltpu.VMEM((_FPT,), jnp.int32),
        pltpu.VMEM((12, _FPT), jnp.float32),
    ],
    compiler_params=_SC_PARAMS,
)


_BQ = 512


_NBLK = _NC // _BQ


def _knn_body(q_ref, tbl_ref, d_ref, i_ref, iota_ref):
    @pl.when(pl.program_id(0) == 0)
    def _():
        iota_ref[...] = lax.broadcasted_iota(
            jnp.int32, (_BQ, _NF), 1).astype(jnp.float32)

    q = q_ref[...]
    qx, qy, qz = q[:, 0:1], q[:, 1:2], q[:, 2:3]
    kt = tbl_ref[0]
    kx, ky, kz = kt[0:1, :], kt[1:2, :], kt[2:3, :]
    psq = qx * qx + qy * qy + qz * qz
    ksq = kx * kx + ky * ky + kz * kz
    g = jnp.dot(q.astype(jnp.bfloat16), kt.astype(jnp.bfloat16),
                preferred_element_type=jnp.float32)
    d = psq + ksq - 2.0 * g
    minv = jnp.min(d, axis=1, keepdims=True)
    idxf = jnp.min(jnp.where(d <= minv, iota_ref[...], jnp.float32(_NF)),
                   axis=1)
    d_ref[...] = minv[:, 0]
    i_ref[...] = idxf.astype(jnp.int32)


_knn_kernel = pl.pallas_call(
    _knn_body,
    grid=(2 * _NBLK,),
    in_specs=[
        pl.BlockSpec((_BQ, 3), lambda i: (i, 0)),
        pl.BlockSpec((1, 3, _NF), lambda i: (i // _NBLK, 0, 0)),
    ],
    out_specs=[
        pl.BlockSpec((_BQ,), lambda i: (i,)),
        pl.BlockSpec((_BQ,), lambda i: (i,)),
    ],
    out_shape=[
        jax.ShapeDtypeStruct((2 * _NC,), jnp.float32),
        jax.ShapeDtypeStruct((2 * _NC,), jnp.int32),
    ],
    scratch_shapes=[pltpu.VMEM((_BQ, _NF), jnp.float32)],
    compiler_params=pltpu.CompilerParams(vmem_limit_bytes=128 * 1024 * 1024),
)


def _friction_body(tbl_hbm, ip_hbm, ic_hbm, dp_hbm, dc_hbm,
                   cpos_hbm, cpred_hbm, m_hbm, out_hbm,
                   tbl_v, ip_v, ic_v, dp_v, dc_v, cpos_v, cpred_v, m_v,
                   acc_v, shared_v, red_v, out_v):
    c = lax.axis_index("c")
    s = lax.axis_index("s")
    wid = s * _NCORES + c
    base = wid * _PPT
    pltpu.sync_copy(tbl_hbm, tbl_v)
    pltpu.sync_copy(ip_hbm.at[pl.ds(base, _PPT)], ip_v)
    pltpu.sync_copy(ic_hbm.at[pl.ds(base, _PPT)], ic_v)
    pltpu.sync_copy(dp_hbm.at[pl.ds(base, _PPT)], dp_v)
    pltpu.sync_copy(dc_hbm.at[pl.ds(base, _PPT)], dc_v)
    pltpu.sync_copy(cpos_hbm.at[:, pl.ds(base, _PPT)], cpos_v)
    pltpu.sync_copy(cpred_hbm.at[:, pl.ds(base, _PPT)], cpred_v)
    pltpu.sync_copy(m_hbm.at[pl.ds(base, _PPT)], m_v)

    def g(row, idx):
        return plsc.load_gather(tbl_v, [jnp.full((16,), row, jnp.int32), idx])

    def unit(n):
        s2 = n[0] * n[0] + n[1] * n[1] + n[2] * n[2]
        inv = 1.0 / jnp.maximum(_sqrt16(s2), 1e-12)
        return [n[0] * inv, n[1] * inv, n[2] * inv]

    acc = jnp.zeros((16,), jnp.float32)
    for k in range(_PPT // 16):
        o = k * 16
        jp = ip_v[pl.ds(o, 16)]
        jc = ic_v[pl.ds(o, 16)]
        pp = [g(r, jp) for r in (0, 1, 2)]
        pc = [g(r, jp) for r in (3, 4, 5)]
        nmu = unit([g(r, jp) for r in (6, 7, 8)])
        ncu = unit([g(r, jc) for r in (9, 10, 11)])
        nm = [(nmu[d] + ncu[d]) * 0.5 for d in range(3)]
        dl = [cpred_v[d, pl.ds(o, 16)] - (cpos_v[d, pl.ds(o, 16)] + (pc[d] - pp[d]))
              for d in range(3)]
        t = dl[0] * nm[0] + dl[1] * nm[1] + dl[2] * nm[2]
        dproj = [dl[d] - t * nm[d] for d in range(3)]
        d2 = dproj[0] * dproj[0] + dproj[1] * dproj[1] + dproj[2] * dproj[2]
        dn = _sqrt16(d2)
        cosn = jnp.abs(dproj[2]) / jnp.maximum(dn, 1e-8)
        cosp = _sqrt16(jnp.abs(1.0 - cosn * cosn) + _EPS)
        mask = jnp.logical_and(dp_v[pl.ds(o, 16)] < _RAD,
                               dc_v[pl.ds(o, 16)] < _RAD)
        fr = m_v[pl.ds(o, 16)] * cosp * _GRAV * dn * jnp.where(mask, 1.0, 0.0)
        acc = acc + fr
    acc_v[...] = acc
    pltpu.sync_copy(acc_v, shared_v.at[s])
    plsc.subcore_barrier()

    @pl.when(s == 0)
    def _():
        pltpu.sync_copy(shared_v, red_v)
        tot = jnp.zeros((16,), jnp.float32)
        for r in range(_NSUB):
            tot = tot + red_v[r]
        out_v[...] = jnp.full((16,), jnp.sum(tot), jnp.float32)
        pltpu.sync_copy(out_v, out_hbm.at[c])


_friction_kernel = pl.kernel(
    _friction_body,
    out_type=jax.ShapeDtypeStruct((_NCORES, 16), jnp.float32),
    mesh=plsc.VectorSubcoreMesh(**_MESH),
    scratch_types=[
        pltpu.VMEM((12, _NF), jnp.float32),
        pltpu.VMEM((_PPT,), jnp.int32),
        pltpu.VMEM((_PPT,), jnp.int32),
        pltpu.VMEM((_PPT,), jnp.float32),
        pltpu.VMEM((_PPT,), jnp.float32),
        pltpu.VMEM((3, _PPT), jnp.float32),
        pltpu.VMEM((3, _PPT), jnp.float32),
        pltpu.VMEM((_PPT,), jnp.float32),
        pltpu.VMEM((16,), jnp.float32),
        pltpu.VMEM_SHARED((_NSUB, 16), jnp.float32),
        pltpu.VMEM((_NSUB, 16), jnp.float32),
        pltpu.VMEM((16,), jnp.float32),
    ],
    compiler_params=_SC_PARAMS,
)


def kernel(cloth_pos, cloth_pred_pos, cloth_v_mass,
           obstacle_prev_pos, obstacle_pos, obstacle_faces):
    vt = jnp.concatenate([obstacle_prev_pos.T, obstacle_pos.T], axis=0)
    tbl = _face_kernel(vt, obstacle_faces[0], obstacle_faces[1],
                       obstacle_faces[2])
    q2 = jnp.concatenate([cloth_pos, cloth_pred_pos], axis=0)
    dist2, idx2 = _knn_kernel(q2, tbl[:6].reshape(2, 3, _NF))
    dp, dc = dist2[:_NC], dist2[_NC:]
    ip, ic = idx2[:_NC], idx2[_NC:]
    part = _friction_kernel(tbl, ip, ic, dp, dc,
                            cloth_pos.T, cloth_pred_pos.T,
                            cloth_v_mass[:, 0])
    return part[0, 0] + part[1, 0]


# speedup vs baseline: 1.1576x; 1.0059x over previous
import jax
import jax.numpy as jnp
from jax import lax
from jax.experimental import pallas as pl
from jax.experimental.pallas import tpu as pltpu
from jax.experimental.pallas import tpu_sc as plsc

_NC = 8192
_NV = 6000
_NF = 8192
_RAD = 0.003
_GRAV = 9.81
_EPS = 1e-4

_NCORES = 2
_NSUB = 16
_NW = _NCORES * _NSUB
_FPT = _NF // _NW
_PPT = _NC // _NW

_MESH = dict(core_axis_name="c", subcore_axis_name="s",
             num_cores=_NCORES, num_subcores=_NSUB)
_SC_PARAMS = pltpu.CompilerParams(use_tc_tiling_on_sc=False,
                                  needs_layout_passes=False)


def _sqrt16(x):
    i = lax.bitcast_convert_type(x, jnp.int32)
    y = lax.bitcast_convert_type(jnp.int32(0x5F3759DF) - (i >> 1), jnp.float32)
    for _ in range(3):
        y = y * (1.5 - 0.5 * x * y * y)
    return x * y


def _face_body(vt_hbm, f0_hbm, f1_hbm, f2_hbm, tbl_hbm,
               vt_v, f0_v, f1_v, f2_v, o_v):
    c = lax.axis_index("c")
    s = lax.axis_index("s")
    wid = s * _NCORES + c
    base = wid * _FPT
    pltpu.sync_copy(vt_hbm, vt_v)
    pltpu.sync_copy(f0_hbm.at[pl.ds(base, _FPT)], f0_v)
    pltpu.sync_copy(f1_hbm.at[pl.ds(base, _FPT)], f1_v)
    pltpu.sync_copy(f2_hbm.at[pl.ds(base, _FPT)], f2_v)

    def g(row, idx):
        return plsc.load_gather(vt_v, [jnp.full((16,), row, jnp.int32), idx])

    for k in range(_FPT // 16):
        o = k * 16
        i0 = f0_v[pl.ds(o, 16)]
        i1 = f1_v[pl.ds(o, 16)]
        i2 = f2_v[pl.ds(o, 16)]
        for t in range(2):
            r0 = 3 * t
            v0 = [g(r0 + d, i0) for d in range(3)]
            v1 = [g(r0 + d, i1) for d in range(3)]
            v2 = [g(r0 + d, i2) for d in range(3)]
            for d in range(3):
                o_v[3 * t + d, pl.ds(o, 16)] = (v0[d] + v1[d] + v2[d]) * (1.0 / 3.0)
            e1 = [v1[d] - v0[d] for d in range(3)]
            e2 = [v2[d] - v0[d] for d in range(3)]
            o_v[6 + r0 + 0, pl.ds(o, 16)] = e1[1] * e2[2] - e1[2] * e2[1]
            o_v[6 + r0 + 1, pl.ds(o, 16)] = e1[2] * e2[0] - e1[0] * e2[2]
            o_v[6 + r0 + 2, pl.ds(o, 16)] = e1[0] * e2[1] - e1[1] * e2[0]
    pltpu.sync_copy(o_v, tbl_hbm.at[:, pl.ds(base, _FPT)])


_face_kernel = pl.kernel(
    _face_body,
    out_type=jax.ShapeDtypeStruct((12, _NF), jnp.float32),
    mesh=plsc.VectorSubcoreMesh(**_MESH),
    scratch_types=[
        pltpu.VMEM((6, _NV), jnp.float32),
        pltpu.VMEM((_FPT,), jnp.int32),
        pltpu.VMEM((_FPT,), jnp.int32),
        pltpu.VMEM((_FPT,), jnp.int32),
        pltpu.VMEM((12, _FPT), jnp.float32),
    ],
    compiler_params=_SC_PARAMS,
)


_BQ = 512


_NBLK = _NC // _BQ


def _knn_body(q_ref, tbl_ref, d_ref, i_ref, iota_ref):
    @pl.when(pl.program_id(0) == 0)
    def _():
        iota_ref[...] = lax.broadcasted_iota(
            jnp.int32, (8, _NF), 1).astype(jnp.float32)

    q = q_ref[...]
    qx, qy, qz = q[:, 0:1], q[:, 1:2], q[:, 2:3]
    kt = tbl_ref[0]
    kx, ky, kz = kt[0:1, :], kt[1:2, :], kt[2:3, :]
    psq = qx * qx + qy * qy + qz * qz
    ksq = kx * kx + ky * ky + kz * kz
    g = jnp.dot(q.astype(jnp.bfloat16), kt.astype(jnp.bfloat16),
                preferred_element_type=jnp.float32)
    d = psq + ksq - 2.0 * g
    minv = jnp.min(d, axis=1, keepdims=True)
    idxf = jnp.min(jnp.where(d <= minv, iota_ref[0:1, :], jnp.float32(_NF)),
                   axis=1)
    d_ref[...] = minv[:, 0]
    i_ref[...] = idxf.astype(jnp.int32)


_knn_kernel = pl.pallas_call(
    _knn_body,
    grid=(2 * _NBLK,),
    in_specs=[
        pl.BlockSpec((_BQ, 3), lambda i: (i, 0)),
        pl.BlockSpec((1, 3, _NF), lambda i: (i // _NBLK, 0, 0)),
    ],
    out_specs=[
        pl.BlockSpec((_BQ,), lambda i: (i,)),
        pl.BlockSpec((_BQ,), lambda i: (i,)),
    ],
    out_shape=[
        jax.ShapeDtypeStruct((2 * _NC,), jnp.float32),
        jax.ShapeDtypeStruct((2 * _NC,), jnp.int32),
    ],
    scratch_shapes=[pltpu.VMEM((8, _NF), jnp.float32)],
    compiler_params=pltpu.CompilerParams(vmem_limit_bytes=128 * 1024 * 1024),
)


def _friction_body(tbl_hbm, ip_hbm, ic_hbm, dp_hbm, dc_hbm,
                   cpos_hbm, cpred_hbm, m_hbm, out_hbm,
                   tbl_v, ip_v, ic_v, dp_v, dc_v, cpos_v, cpred_v, m_v,
                   acc_v, shared_v, red_v, out_v):
    c = lax.axis_index("c")
    s = lax.axis_index("s")
    wid = s * _NCORES + c
    base = wid * _PPT
    pltpu.sync_copy(tbl_hbm, tbl_v)
    pltpu.sync_copy(ip_hbm.at[pl.ds(base, _PPT)], ip_v)
    pltpu.sync_copy(ic_hbm.at[pl.ds(base, _PPT)], ic_v)
    pltpu.sync_copy(dp_hbm.at[pl.ds(base, _PPT)], dp_v)
    pltpu.sync_copy(dc_hbm.at[pl.ds(base, _PPT)], dc_v)
    pltpu.sync_copy(cpos_hbm.at[:, pl.ds(base, _PPT)], cpos_v)
    pltpu.sync_copy(cpred_hbm.at[:, pl.ds(base, _PPT)], cpred_v)
    pltpu.sync_copy(m_hbm.at[pl.ds(base, _PPT)], m_v)

    def g(row, idx):
        return plsc.load_gather(tbl_v, [jnp.full((16,), row, jnp.int32), idx])

    def unit(n):
        s2 = n[0] * n[0] + n[1] * n[1] + n[2] * n[2]
        inv = 1.0 / jnp.maximum(_sqrt16(s2), 1e-12)
        return [n[0] * inv, n[1] * inv, n[2] * inv]

    acc = jnp.zeros((16,), jnp.float32)
    for k in range(_PPT // 16):
        o = k * 16
        jp = ip_v[pl.ds(o, 16)]
        jc = ic_v[pl.ds(o, 16)]
        pp = [g(r, jp) for r in (0, 1, 2)]
        pc = [g(r, jp) for r in (3, 4, 5)]
        nmu = unit([g(r, jp) for r in (6, 7, 8)])
        ncu = unit([g(r, jc) for r in (9, 10, 11)])
        nm = [(nmu[d] + ncu[d]) * 0.5 for d in range(3)]
        dl = [cpred_v[d, pl.ds(o, 16)] - (cpos_v[d, pl.ds(o, 16)] + (pc[d] - pp[d]))
              for d in range(3)]
        t = dl[0] * nm[0] + dl[1] * nm[1] + dl[2] * nm[2]
        dproj = [dl[d] - t * nm[d] for d in range(3)]
        d2 = dproj[0] * dproj[0] + dproj[1] * dproj[1] + dproj[2] * dproj[2]
        dn = _sqrt16(d2)
        cosn = jnp.abs(dproj[2]) / jnp.maximum(dn, 1e-8)
        cosp = _sqrt16(jnp.abs(1.0 - cosn * cosn) + _EPS)
        mask = jnp.logical_and(dp_v[pl.ds(o, 16)] < _RAD,
                               dc_v[pl.ds(o, 16)] < _RAD)
        fr = m_v[pl.ds(o, 16)] * cosp * _GRAV * dn * jnp.where(mask, 1.0, 0.0)
        acc = acc + fr
    acc_v[...] = acc
    pltpu.sync_copy(acc_v, shared_v.at[s])
    plsc.subcore_barrier()

    @pl.when(s == 0)
    def _():
        pltpu.sync_copy(shared_v, red_v)
        tot = jnp.zeros((16,), jnp.float32)
        for r in range(_NSUB):
            tot = tot + red_v[r]
        out_v[...] = jnp.full((16,), jnp.sum(tot), jnp.float32)
        pltpu.sync_copy(out_v, out_hbm.at[c])


_friction_kernel = pl.kernel(
    _friction_body,
    out_type=jax.ShapeDtypeStruct((_NCORES, 16), jnp.float32),
    mesh=plsc.VectorSubcoreMesh(**_MESH),
    scratch_types=[
        pltpu.VMEM((12, _NF), jnp.float32),
        pltpu.VMEM((_PPT,), jnp.int32),
        pltpu.VMEM((_PPT,), jnp.int32),
        pltpu.VMEM((_PPT,), jnp.float32),
        pltpu.VMEM((_PPT,), jnp.float32),
        pltpu.VMEM((3, _PPT), jnp.float32),
        pltpu.VMEM((3, _PPT), jnp.float32),
        pltpu.VMEM((_PPT,), jnp.float32),
        pltpu.VMEM((16,), jnp.float32),
        pltpu.VMEM_SHARED((_NSUB, 16), jnp.float32),
        pltpu.VMEM((_NSUB, 16), jnp.float32),
        pltpu.VMEM((16,), jnp.float32),
    ],
    compiler_params=_SC_PARAMS,
)


def kernel(cloth_pos, cloth_pred_pos, cloth_v_mass,
           obstacle_prev_pos, obstacle_pos, obstacle_faces):
    vt = jnp.concatenate([obstacle_prev_pos.T, obstacle_pos.T], axis=0)
    tbl = _face_kernel(vt, obstacle_faces[0], obstacle_faces[1],
                       obstacle_faces[2])
    q2 = jnp.concatenate([cloth_pos, cloth_pred_pos], axis=0)
    dist2, idx2 = _knn_kernel(q2, tbl[:6].reshape(2, 3, _NF))
    dp, dc = dist2[:_NC], dist2[_NC:]
    ip, ic = idx2[:_NC], idx2[_NC:]
    part = _friction_kernel(tbl, ip, ic, dp, dc,
                            cloth_pos.T, cloth_pred_pos.T,
                            cloth_v_mass[:, 0])
    return part[0, 0] + part[1, 0]
